# baseline (speedup 1.0000x reference)
import functools

import jax
import jax.numpy as jnp
from jax import lax
from jax.experimental import pallas as pl
from jax.experimental.pallas import tpu as pltpu
from jax.experimental.pallas import tpu_sc as plsc

_NC = 2
_NS = 16
_LANES = 16

_CHUNK = 1024


def _vtake(v, ids):
  return lax.gather(
      v, ids[:, None],
      dimension_numbers=lax.GatherDimensionNumbers(
          offset_dims=(), collapsed_slice_dims=(0,), start_index_map=(0,)),
      slice_sizes=(1,),
      mode=lax.GatherScatterMode.PROMISE_IN_BOUNDS)


def _sc_gather(args, B, V, D, n_steps):
  assert n_steps % 2 == 0 and n_steps >= 4
  mesh = plsc.VectorSubcoreMesh(core_axis_name="c", subcore_axis_name="s")
  step_elems = _NS * _CHUNK * D

  @functools.partial(
      pl.kernel,
      out_type=jax.ShapeDtypeStruct((B * D,), jnp.float32),
      mesh=mesh,
      scratch_types=[
          pltpu.VMEM((V * D,), jnp.float32),
          pltpu.VMEM((2, _CHUNK), jnp.int32),
          pltpu.VMEM((_CHUNK * D,), jnp.float32),
          pltpu.VMEM((_LANES,), jnp.int32),
          pltpu.VMEM_SHARED((2, step_elems), jnp.float32),
          pltpu.SemaphoreType.DMA,
          pltpu.SemaphoreType.DMA,
          pltpu.SemaphoreType.DMA,
          pltpu.SemaphoreType.DMA,
      ],
      compiler_params=pltpu.CompilerParams(use_tc_tiling_on_sc=False,
                                           needs_layout_passes=False),
  )
  def k(idx_hbm, off_hbm, table_hbm, out_hbm, table_v, idx_v, rows_v, off_v,
        stage_s, sem_i0, sem_i1, sem_d0, sem_d1):
    s_i = lax.axis_index("s")
    c_i = lax.axis_index("c")
    sc_base = c_i * (B // _NC)
    sem_i = (sem_i0, sem_i1)
    sem_d = (sem_d0, sem_d1)
    pltpu.sync_copy(off_hbm, off_v)
    pltpu.sync_copy(table_hbm, table_v)
    offv = off_v[...] * D
    iota = lax.iota(jnp.int32, _LANES)
    oi = offv + iota
    consts = [jnp.full((_LANES,), h * _LANES, dtype=jnp.int32)
              for h in range(D // _LANES)]

    def chunk_row(t):
      return sc_base + (t * _NS + s_i) * _CHUNK

    def gather_group(p):
      iv = idx_v.at[p]
      rv = rows_v

      def body(c4, carry):
        idxv = plsc.load_gather(iv, [c4 * _LANES + iota])
        gs = []
        for r in range(_LANES):
          ridx = _vtake(idxv, jnp.full((_LANES,), r, dtype=jnp.int32))
          e0 = ridx * D + oi
          for h in range(D // _LANES):
            gs.append(plsc.load_gather(table_v, [e0 + consts[h]]))
        base_w = c4 * (_LANES * D)
        for j, g in enumerate(gs):
          rv[pl.ds(base_w + j * _LANES, _LANES)] = g
        return carry

      lax.fori_loop(0, _CHUNK // _LANES, body, None)

    pltpu.sync_copy(idx_hbm.at[pl.ds(chunk_row(0), _CHUNK)], idx_v.at[0])
    pltpu.async_copy(idx_hbm.at[pl.ds(chunk_row(1), _CHUNK)], idx_v.at[1],
                     sem_i[1])

    def half(t2, q):
      t = t2 * 2 + q
      rv = rows_v
      out_step = out_hbm.at[pl.ds((sc_base + t * _NS * _CHUNK) * D,
                                  step_elems)]

      @pl.when(jnp.logical_or(t2 >= 1, q == 1))
      def _():
        pltpu.make_async_copy(idx_hbm.at[pl.ds(chunk_row(t), _CHUNK)],
                              idx_v.at[q], sem_i[q]).wait()
      gather_group(q)

      @pl.when(t2 < n_steps // 2 - 1)
      def _():
        pltpu.async_copy(idx_hbm.at[pl.ds(chunk_row(t + 2), _CHUNK)],
                         idx_v.at[q], sem_i[q])

      @pl.when(jnp.logical_and(s_i == 0, t2 >= 1))
      def _():
        pltpu.make_async_copy(stage_s.at[q], out_step, sem_d[q]).wait()
      plsc.subcore_barrier()

      pltpu.sync_copy(rv, stage_s.at[q].at[pl.ds(s_i * (_CHUNK * D),
                                                 _CHUNK * D)])
      plsc.subcore_barrier()

      @pl.when(s_i == 0)
      def _():
        pltpu.async_copy(stage_s.at[q], out_step, sem_d[q])

    def pair(t2, carry):
      half(t2, 0)
      half(t2, 1)
      return carry

    lax.fori_loop(0, n_steps // 2, pair, None)

    @pl.when(s_i == 0)
    def _():
      for q, t in ((0, n_steps - 2), (1, n_steps - 1)):
        out_step = out_hbm.at[pl.ds((sc_base + t * _NS * _CHUNK) * D,
                                    step_elems)]
        pltpu.make_async_copy(stage_s.at[q], out_step, sem_d[q]).wait()

  idx, off_vec, table = args
  return k(idx, off_vec, table)


def kernel(rel_pos_matrix, len, embeddings_table):
  n = rel_pos_matrix.shape[0]
  V, D = embeddings_table.shape
  B = n * n
  assert D % _LANES == 0
  idx = rel_pos_matrix.reshape(B)
  off = jnp.asarray(len, jnp.int32) - jnp.int32(n)
  off_vec = jnp.full((_LANES,), off, dtype=jnp.int32)
  per_sc = B // _NC
  assert per_sc % (_NS * _CHUNK) == 0
  n_steps = per_sc // (_NS * _CHUNK)
  out = _sc_gather((idx, off_vec, embeddings_table.reshape(V * D)), B, V, D,
                   n_steps)
  return out.reshape(n, n, D)


# speedup vs baseline: 1.0719x; 1.0719x over previous
import functools

import jax
import jax.numpy as jnp
from jax import lax
from jax.experimental import pallas as pl
from jax.experimental.pallas import tpu as pltpu
from jax.experimental.pallas import tpu_sc as plsc

_NC = 2
_NS = 16
_NW = _NC * _NS
_LANES = 16

_CHUNK = 512
_NBUF = 4


def _vtake(v, ids):
  return lax.gather(
      v, ids[:, None],
      dimension_numbers=lax.GatherDimensionNumbers(
          offset_dims=(), collapsed_slice_dims=(0,), start_index_map=(0,)),
      slice_sizes=(1,),
      mode=lax.GatherScatterMode.PROMISE_IN_BOUNDS)


def _sc_gather(args, B, V, D, n_groups):
  per_w = B // _NW
  assert n_groups % _NBUF == 0 and n_groups >= 2 * _NBUF
  mesh = plsc.VectorSubcoreMesh(core_axis_name="c", subcore_axis_name="s")

  @functools.partial(
      pl.kernel,
      out_type=jax.ShapeDtypeStruct((B * D,), jnp.float32),
      mesh=mesh,
      scratch_types=[
          pltpu.VMEM((V * D,), jnp.float32),
          pltpu.VMEM((_NBUF, _CHUNK), jnp.int32),
          pltpu.VMEM((_NBUF, _CHUNK * D), jnp.float32),
          pltpu.VMEM((_LANES,), jnp.int32),
      ] + [pltpu.SemaphoreType.DMA] * (2 * _NBUF),
      compiler_params=pltpu.CompilerParams(use_tc_tiling_on_sc=False,
                                           needs_layout_passes=False),
  )
  def k(idx_hbm, off_hbm, table_hbm, out_hbm, table_v, idx_v, rows_v, off_v,
        *sems):
    wid = lax.axis_index("s") * _NC + lax.axis_index("c")
    base = wid * per_w
    sem_i = sems[:_NBUF]
    sem_o = sems[_NBUF:]
    pltpu.sync_copy(off_hbm, off_v)
    pltpu.sync_copy(table_hbm, table_v)
    offv = off_v[...] * D
    iota = lax.iota(jnp.int32, _LANES)
    oi = offv + iota
    consts = [jnp.full((_LANES,), h * _LANES, dtype=jnp.int32)
              for h in range(D // _LANES)]

    for p in range(_NBUF):
      pltpu.async_copy(idx_hbm.at[pl.ds(base + p * _CHUNK, _CHUNK)],
                       idx_v.at[p], sem_i[p])

    def gather_group(p):
      iv = idx_v.at[p]
      rv = rows_v.at[p]

      def body(c4, carry):
        idxv = plsc.load_gather(iv, [c4 * _LANES + iota])
        gs = []
        for r in range(_LANES):
          ridx = _vtake(idxv, jnp.full((_LANES,), r, dtype=jnp.int32))
          e0 = ridx * D + oi
          for h in range(D // _LANES):
            gs.append(plsc.load_gather(table_v, [e0 + consts[h]]))
        base_w = c4 * (_LANES * D)
        for j, g in enumerate(gs):
          rv[pl.ds(base_w + j * _LANES, _LANES)] = g
        return carry

      lax.fori_loop(0, _CHUNK // _LANES, body, None)

    def half(g2, p):
      g = g2 * _NBUF + p
      start = base + g * _CHUNK
      rv = rows_v.at[p]
      out_slice = out_hbm.at[pl.ds(start * D, _CHUNK * D)]

      @pl.when(g2 >= 1)
      def _():
        pltpu.make_async_copy(rv, out_slice, sem_o[p]).wait()

      pltpu.make_async_copy(idx_hbm.at[pl.ds(start, _CHUNK)], idx_v.at[p],
                            sem_i[p]).wait()
      gather_group(p)

      pltpu.async_copy(rv, out_slice, sem_o[p])

      @pl.when(g2 < n_groups // _NBUF - 1)
      def _():
        pltpu.async_copy(idx_hbm.at[pl.ds(start + _NBUF * _CHUNK, _CHUNK)],
                         idx_v.at[p], sem_i[p])

    def ring(g2, carry):
      for p in range(_NBUF):
        half(g2, p)
      return carry

    lax.fori_loop(0, n_groups // _NBUF, ring, None)

    tail = base + (n_groups - _NBUF) * _CHUNK
    for p in range(_NBUF):
      pltpu.make_async_copy(
          rows_v.at[p],
          out_hbm.at[pl.ds((tail + p * _CHUNK) * D, _CHUNK * D)],
          sem_o[p]).wait()

  idx, off_vec, table = args
  return k(idx, off_vec, table)


def kernel(rel_pos_matrix, len, embeddings_table):
  n = rel_pos_matrix.shape[0]
  V, D = embeddings_table.shape
  B = n * n
  idx = rel_pos_matrix.reshape(B)
  off = jnp.asarray(len, jnp.int32) - jnp.int32(n)
  off_vec = jnp.full((_LANES,), off, dtype=jnp.int32)
  per_w = B // _NW
  assert per_w % _CHUNK == 0
  out = _sc_gather((idx, off_vec, embeddings_table.reshape(V * D)), B, V, D,
                   per_w // _CHUNK)
  return out.reshape(n, n, D)


# speedup vs baseline: 1.0735x; 1.0015x over previous
import functools

import jax
import jax.numpy as jnp
from jax import lax
from jax.experimental import pallas as pl
from jax.experimental.pallas import tpu as pltpu
from jax.experimental.pallas import tpu_sc as plsc

_NC = 2
_NS = 16
_NW = _NC * _NS
_LANES = 16

_CHUNK = 1024


def _vtake(v, ids):
  return lax.gather(
      v, ids[:, None],
      dimension_numbers=lax.GatherDimensionNumbers(
          offset_dims=(), collapsed_slice_dims=(0,), start_index_map=(0,)),
      slice_sizes=(1,),
      mode=lax.GatherScatterMode.PROMISE_IN_BOUNDS)


def _sc_gather(args, B, V, D, n_groups):
  per_w = B // _NW
  assert n_groups % 2 == 0 and n_groups >= 4
  mesh = plsc.VectorSubcoreMesh(core_axis_name="c", subcore_axis_name="s")

  @functools.partial(
      pl.kernel,
      out_type=jax.ShapeDtypeStruct((B * D,), jnp.float32),
      mesh=mesh,
      scratch_types=[
          pltpu.VMEM((V * D,), jnp.float32),
          pltpu.VMEM((2, _CHUNK), jnp.int32),
          pltpu.VMEM((2, _CHUNK * D), jnp.float32),
          pltpu.VMEM((_LANES,), jnp.int32),
          pltpu.SemaphoreType.DMA,
          pltpu.SemaphoreType.DMA,
          pltpu.SemaphoreType.DMA,
          pltpu.SemaphoreType.DMA,
      ],
      compiler_params=pltpu.CompilerParams(use_tc_tiling_on_sc=False,
                                           needs_layout_passes=False),
  )
  def k(idx_hbm, off_hbm, table_hbm, out_hbm, table_v, idx_v, rows_v, off_v,
        sem_i0, sem_i1, sem_o0, sem_o1):
    wid = lax.axis_index("s") * _NC + lax.axis_index("c")
    base = wid * per_w
    sem_i = (sem_i0, sem_i1)
    sem_o = (sem_o0, sem_o1)
    pltpu.sync_copy(off_hbm, off_v)
    pltpu.sync_copy(table_hbm, table_v)
    offv = off_v[...] * D
    iota = lax.iota(jnp.int32, _LANES)
    oi = offv + iota
    consts = [jnp.full((_LANES,), h * _LANES, dtype=jnp.int32)
              for h in range(D // _LANES)]

    pltpu.async_copy(idx_hbm.at[pl.ds(base, _CHUNK)], idx_v.at[0], sem_i[0])
    pltpu.async_copy(idx_hbm.at[pl.ds(base + _CHUNK, _CHUNK)], idx_v.at[1],
                     sem_i[1])

    def gather_group(p):
      iv = idx_v.at[p]
      rv = rows_v.at[p]

      def body(c4, carry):
        idxv = plsc.load_gather(iv, [c4 * _LANES + iota])
        gs = []
        for r in range(_LANES):
          ridx = _vtake(idxv, jnp.full((_LANES,), r, dtype=jnp.int32))
          e0 = ridx * D + oi
          for h in range(D // _LANES):
            gs.append(plsc.load_gather(table_v, [e0 + consts[h]]))
        base_w = c4 * (_LANES * D)
        for j, g in enumerate(gs):
          rv[pl.ds(base_w + j * _LANES, _LANES)] = g
        return carry

      lax.fori_loop(0, _CHUNK // _LANES, body, None)

    def half(g2, p):
      g = g2 * 2 + p
      start = base + g * _CHUNK
      rv = rows_v.at[p]
      out_slice = out_hbm.at[pl.ds(start * D, _CHUNK * D)]

      @pl.when(g2 >= 1)
      def _():
        pltpu.make_async_copy(rv, out_slice, sem_o[p]).wait()

      pltpu.make_async_copy(idx_hbm.at[pl.ds(start, _CHUNK)], idx_v.at[p],
                            sem_i[p]).wait()
      gather_group(p)

      pltpu.async_copy(rv, out_slice, sem_o[p])

      @pl.when(g2 < n_groups // 2 - 1)
      def _():
        pltpu.async_copy(idx_hbm.at[pl.ds(start + 2 * _CHUNK, _CHUNK)],
                         idx_v.at[p], sem_i[p])

    def pair(g2, carry):
      half(g2, 0)
      half(g2, 1)
      return carry

    lax.fori_loop(0, n_groups // 2, pair, None)

    tail = base + (n_groups - 2) * _CHUNK
    pltpu.make_async_copy(rows_v.at[0],
                          out_hbm.at[pl.ds(tail * D, _CHUNK * D)],
                          sem_o[0]).wait()
    pltpu.make_async_copy(rows_v.at[1],
                          out_hbm.at[pl.ds((tail + _CHUNK) * D, _CHUNK * D)],
                          sem_o[1]).wait()

  idx, off_vec, table = args
  return k(idx, off_vec, table)


def kernel(rel_pos_matrix, len, embeddings_table):
  n = rel_pos_matrix.shape[0]
  V, D = embeddings_table.shape
  B = n * n
  idx = rel_pos_matrix.reshape(B)
  off = jnp.asarray(len, jnp.int32) - jnp.int32(n)
  off_vec = jnp.full((_LANES,), off, dtype=jnp.int32)
  per_w = B // _NW
  assert per_w % _CHUNK == 0
  out = _sc_gather((idx, off_vec, embeddings_table.reshape(V * D)), B, V, D,
                   per_w // _CHUNK)
  return out.reshape(n, n, D)
